# baseline (speedup 1.0000x reference)
import numpy as np
import jax
import jax.numpy as jnp
from jax import lax
from jax.experimental import pallas as pl
from jax.experimental.pallas import tpu as pltpu
from jax.experimental.pallas import tpu_sc as plsc

N = 10000
E = 320000
D = 128
NC = 2
NS = 16
NW = NC * NS
C = 80
WMAX = 23040
NCHUNK = WMAX // C
ACC_ROWS = 10112
SUB_ROWS = ACC_ROWS // NS

_QUOTAS = ([10080] * 11 + [9840] * 4 + [9760]) * 2
_BOUNDS = np.concatenate([[0], np.cumsum(_QUOTAS)])
_SIZES = np.diff(_BOUNDS)
_WID = np.repeat(np.arange(NW), _SIZES)
_PSTART = np.repeat(_BOUNDS[:-1], _SIZES)
_GARBAGE = N + 64 + np.arange(NW)
_ARANGE = np.arange(E)
_PMASK = np.zeros(E, bool)
_PMASK[_BOUNDS[:-1]] = True
_DIVS = (80, 40, 20, 10, 5, 2, 1)
_FILL = ((np.arange(NW * WMAX) % N) |
         (np.repeat(_GARBAGE, WMAX) << 16)).astype(np.int32)


def _segsum_body(h_hbm, pk, zeros_hbm, out_hbm,
                 pk_v, src_v, dst_v, rows_v, acc, sem):
    c = lax.axis_index("c")
    s = lax.axis_index("s")

    row0 = s * SUB_ROWS
    pltpu.sync_copy(zeros_hbm.at[pl.ds(row0, SUB_ROWS)],
                    acc.at[pl.ds(row0, SUB_ROWS)])

    pltpu.sync_copy(pk.at[c, s], pk_v)
    plsc.subcore_barrier()

    def body(j, carry):
        for g in range(C // 16):
            v = pk_v[j, pl.ds(16 * g, 16)]
            src_v[0, pl.ds(16 * g, 16)] = v & 0xFFFF
            dst_v[0, pl.ds(16 * g, 16)] = lax.shift_right_logical(v, 16)
        pltpu.async_copy(h_hbm.at[src_v.at[0]], rows_v, sem).wait()
        pltpu.sync_copy(rows_v, acc.at[dst_v.at[0]], add=True)
        return carry

    lax.fori_loop(0, NCHUNK, body, 0)
    plsc.subcore_barrier()

    pltpu.sync_copy(acc.at[pl.ds(row0, SUB_ROWS)],
                    out_hbm.at[c, pl.ds(row0, SUB_ROWS)])


_segsum_sc = pl.kernel(
    _segsum_body,
    out_type=jax.ShapeDtypeStruct((NC, ACC_ROWS, D), jnp.float32),
    mesh=plsc.VectorSubcoreMesh(core_axis_name="c", subcore_axis_name="s"),
    scratch_types=[
        pltpu.VMEM((NCHUNK, C), jnp.int32),
        pltpu.VMEM((1, C), jnp.int32),
        pltpu.VMEM((1, C), jnp.int32),
        pltpu.VMEM((C, D), jnp.float32),
        pltpu.VMEM_SHARED((ACC_ROWS, D), jnp.float32),
        pltpu.SemaphoreType.DMA,
    ],
)


def _merge_mm1_body(ids_ref, eps_ref, base_ref, slots_ref, h_ref, w1_ref,
                    out_ref, pooled_ref):
    pooled_ref[...] = base_ref[...]

    def body(k, carry):
        row = ids_ref[k]
        cur = pooled_ref[pl.ds(row, 1), :]
        pooled_ref[pl.ds(row, 1), :] = cur + slots_ref[pl.ds(k, 1), :]
        return carry

    lax.fori_loop(0, 2 * NW, body, 0)
    pooled = pooled_ref[...] + (1.0 + eps_ref[0]) * h_ref[...]
    out_ref[...] = jnp.dot(pooled, w1_ref[...],
                           preferred_element_type=jnp.float32)


def _merge_mm1(ids, eps_l, base, slots, h, w1):
    return pl.pallas_call(
        _merge_mm1_body,
        out_shape=jax.ShapeDtypeStruct((N, D), jnp.float32),
        in_specs=[pl.BlockSpec(memory_space=pltpu.SMEM)] * 2 +
                 [pl.BlockSpec(memory_space=pltpu.VMEM)] * 4,
        out_specs=pl.BlockSpec(memory_space=pltpu.VMEM),
        scratch_shapes=[pltpu.VMEM((N, D), jnp.float32)],
    )(ids, eps_l, base, slots, h, w1)


def kernel(x, edge_index, eps, W1, b1, W2, b2, gamma, beta):
    L = W1.shape[0]
    src, dst = edge_index[0], edge_index[1]

    ar = jnp.asarray(_ARANGE)
    order = jnp.argsort(dst, stable=True)
    src_s = src[order]
    dst_s = dst[order]
    first_dst = dst_s[jnp.asarray(_BOUNDS[:-1])]
    last_dst = dst_s[jnp.asarray(_BOUNDS[1:] - 1)]
    wid = jnp.asarray(_WID)
    is_head = dst_s == first_dst[wid]
    is_tail = dst_s == last_dst[wid]
    slot = jnp.where(is_head, N + 2 * wid,
                     jnp.where(is_tail, N + 2 * wid + 1, dst_s))

    prev = jnp.concatenate([jnp.full((1,), -1, slot.dtype), slot[:-1]])
    start = jnp.asarray(_PMASK) | (slot != prev)
    run_start = jax.lax.cummax(jnp.where(start, ar, 0))
    start2 = start | ((ar - run_start) % C == 0)
    run_start2 = jax.lax.cummax(jnp.where(start2, ar, 0))
    nxt = jnp.flip(jax.lax.cummin(jnp.flip(jnp.where(start2, ar, E))))
    nxt = jnp.concatenate([nxt[1:], jnp.full((1,), E, nxt.dtype)])
    run_end = nxt[run_start2]
    rlen = run_end - run_start2
    dcls = jnp.select([rlen <= d for d in _DIVS[::-1][:-1]],
                      list(_DIVS[::-1][:-1]), _DIVS[0])
    pstart = jnp.asarray(_PSTART)
    base = jnp.zeros(NW, jnp.int32)
    flat = jnp.zeros(E, jnp.int32)
    for d in _DIVS:
        base = -(-base // d) * d
        m = (start2 & (dcls == d)).astype(jnp.int32)
        cum0 = jnp.cumsum(m) - m
        cnt = (cum0[jnp.asarray(_BOUNDS[1:] - 1)] +
               m[jnp.asarray(_BOUNDS[1:] - 1)] -
               cum0[jnp.asarray(_BOUNDS[:-1])])
        rank = cum0[run_start2] - cum0[pstart]
        sel = dcls == d
        flat = jnp.where(
            sel,
            wid * WMAX + base[wid] + rank * d + (ar - run_start2),
            flat)
        base = base + cnt.astype(jnp.int32) * d
    packed = (src_s.astype(jnp.int32) |
              (slot.astype(jnp.int32) << 16))
    pk = jnp.asarray(_FILL).at[flat].set(packed)
    pk = pk.reshape(NC, NS, NCHUNK, C)
    ids = jnp.stack([first_dst, last_dst], axis=1).reshape(2 * NW)
    ids = ids.astype(jnp.int32)
    zeros = jnp.zeros((ACC_ROWS, D), jnp.float32)

    h = x
    outs = [h]
    for l in range(L):
        p = _segsum_sc(h, pk, zeros)
        base_p = p[0, :N] + p[1, :N]
        slots = p[0, N:N + 2 * NW] + p[1, N:N + 2 * NW]
        t = _merge_mm1(ids, eps[l].reshape(1), base_p, slots, h, W1[l])
        t = jnp.maximum(t + b1[l], 0.0)
        t = t @ W2[l] + b2[l]
        mu = jnp.mean(t, axis=0)
        var = jnp.var(t, axis=0)
        t = gamma[l] * (t - mu) * lax.rsqrt(var + 1e-5) + beta[l]
        h = jnp.maximum(t, 0.0)
        outs.append(h)
    return jnp.stack(outs, axis=0)


# speedup vs baseline: 2.6246x; 2.6246x over previous
import numpy as np
import jax
import jax.numpy as jnp
from jax import lax
from jax.experimental import pallas as pl
from jax.experimental.pallas import tpu as pltpu
from jax.experimental.pallas import tpu_sc as plsc

N = 10000
E = 320000
D = 128
NC = 2
NS = 16
NW = NC * NS
C = 80
WMAX = 10080
NCHUNK = WMAX // C
ACC_ROWS = 10240
SUB_ROWS = ACC_ROWS // NS

_QUOTAS = ([10080] * 11 + [9840] * 4 + [9760]) * 2
_BOUNDS = np.concatenate([[0], np.cumsum(_QUOTAS)])
_SIZES = np.diff(_BOUNDS)
_WID = np.repeat(np.arange(NW), _SIZES)
_POS = np.minimum(_BOUNDS[:-1, None] + np.arange(WMAX)[None, :], E - 1)
_VALID = np.arange(WMAX)[None, :] < _SIZES[:, None]
_GARBAGE = N + 64 + np.arange(NW)


def _segsum_body(h_hbm, srcr, dstr, zeros_hbm, out_hbm,
                 src_v, dst_v, rows_v, acc, sem):
    c = lax.axis_index("c")
    s = lax.axis_index("s")

    row0 = s * SUB_ROWS
    pltpu.sync_copy(zeros_hbm.at[pl.ds(row0, SUB_ROWS)],
                    acc.at[pl.ds(row0, SUB_ROWS)])

    pltpu.sync_copy(srcr.at[c, s], src_v)
    pltpu.sync_copy(dstr.at[c, s], dst_v)
    plsc.subcore_barrier()

    def body(j, carry):
        pltpu.async_copy(h_hbm.at[src_v.at[j]], rows_v, sem).wait()
        pltpu.sync_copy(rows_v, acc.at[dst_v.at[j]], add=True)
        return carry

    lax.fori_loop(0, NCHUNK, body, 0)
    plsc.subcore_barrier()

    pltpu.sync_copy(acc.at[pl.ds(row0, SUB_ROWS)],
                    out_hbm.at[c, pl.ds(row0, SUB_ROWS)])


_segsum_sc = pl.kernel(
    _segsum_body,
    out_type=jax.ShapeDtypeStruct((NC, ACC_ROWS, D), jnp.float32),
    mesh=plsc.VectorSubcoreMesh(core_axis_name="c", subcore_axis_name="s"),
    scratch_types=[
        pltpu.VMEM((NCHUNK, C), jnp.int32),
        pltpu.VMEM((NCHUNK, C), jnp.int32),
        pltpu.VMEM((C, D), jnp.float32),
        pltpu.VMEM_SHARED((ACC_ROWS, D), jnp.float32),
        pltpu.SemaphoreType.DMA,
    ],
)


def _merge_mm1_body(ids_ref, eps_ref, base_ref, slots_ref, h_ref, w1_ref,
                    out_ref, pooled_ref):
    pooled_ref[...] = base_ref[...]

    def body(k, carry):
        row = ids_ref[k]
        cur = pooled_ref[pl.ds(row, 1), :]
        pooled_ref[pl.ds(row, 1), :] = cur + slots_ref[pl.ds(k, 1), :]
        return carry

    lax.fori_loop(0, 2 * NW, body, 0)
    pooled = pooled_ref[...] + (1.0 + eps_ref[0]) * h_ref[...]
    out_ref[...] = jnp.dot(pooled, w1_ref[...],
                           preferred_element_type=jnp.float32)


def _merge_mm1(ids, eps_l, base, slots, h, w1):
    return pl.pallas_call(
        _merge_mm1_body,
        out_shape=jax.ShapeDtypeStruct((N, D), jnp.float32),
        in_specs=[pl.BlockSpec(memory_space=pltpu.SMEM)] * 2 +
                 [pl.BlockSpec(memory_space=pltpu.VMEM)] * 4,
        out_specs=pl.BlockSpec(memory_space=pltpu.VMEM),
        scratch_shapes=[pltpu.VMEM((N, D), jnp.float32)],
    )(ids, eps_l, base, slots, h, w1)


def kernel(x, edge_index, eps, W1, b1, W2, b2, gamma, beta):
    L = W1.shape[0]
    src, dst = edge_index[0], edge_index[1]

    order = jnp.argsort(dst, stable=True)
    src_s = src[order]
    dst_s = dst[order]
    first_dst = dst_s[jnp.asarray(_BOUNDS[:-1])]
    last_dst = dst_s[jnp.asarray(_BOUNDS[1:] - 1)]
    wid = jnp.asarray(_WID)
    is_head = dst_s == first_dst[wid]
    is_tail = dst_s == last_dst[wid]
    slot = jnp.where(is_head, N + 2 * wid,
                     jnp.where(is_tail, N + 2 * wid + 1, dst_s))
    pos = jnp.asarray(_POS)
    valid = jnp.asarray(_VALID)
    srcp = jnp.where(valid, src_s[pos], 0)
    dstp = jnp.where(valid, slot[pos], jnp.asarray(_GARBAGE)[:, None])
    srcr = srcp.reshape(NC, NS, NCHUNK, C).astype(jnp.int32)
    dstr = dstp.reshape(NC, NS, NCHUNK, C).astype(jnp.int32)
    ids = jnp.stack([first_dst, last_dst], axis=1).reshape(2 * NW)
    ids = ids.astype(jnp.int32)
    zeros = jnp.zeros((ACC_ROWS, D), jnp.float32)

    h = x
    outs = [h]
    for l in range(L):
        p = _segsum_sc(h, srcr, dstr, zeros)
        base = p[0, :N] + p[1, :N]
        slots = p[0, N:N + 2 * NW] + p[1, N:N + 2 * NW]
        t = _merge_mm1(ids, eps[l].reshape(1), base, slots, h, W1[l])
        t = jnp.maximum(t + b1[l], 0.0)
        t = t @ W2[l] + b2[l]
        mu = jnp.mean(t, axis=0)
        var = jnp.var(t, axis=0)
        t = gamma[l] * (t - mu) * lax.rsqrt(var + 1e-5) + beta[l]
        h = jnp.maximum(t, 0.0)
        outs.append(h)
    return jnp.stack(outs, axis=0)
